# baseline (speedup 1.0000x reference)
import functools

import jax
import jax.numpy as jnp
from jax import lax
from jax.experimental import pallas as pl
from jax.experimental.pallas import tpu as pltpu
from jax.experimental.pallas import tpu_sc as plsc

_NC = 2
_NS = 16
_NW = _NC * _NS
_L = 16


def _sc_gather_pool(user_table, item_table, user_id, song_id,
                    hist_ids, hist_weights):
    B = user_id.shape[0]
    B_, HP = hist_ids.shape
    H = 50
    HG = 56
    E = user_table.shape[1]
    b_per_w = B // _NW
    CB = 8
    CH = CB * H
    n_chunks = b_per_w // CB

    mesh = plsc.VectorSubcoreMesh(core_axis_name="c", subcore_axis_name="s")

    @functools.partial(
        pl.kernel,
        mesh=mesh,
        compiler_params=pltpu.CompilerParams(use_tc_tiling_on_sc=False,
                                             needs_layout_passes=False),
        out_type=[
            jax.ShapeDtypeStruct((B, E), jnp.float32),
            jax.ShapeDtypeStruct((B, E), jnp.float32),
            jax.ShapeDtypeStruct((B * E,), jnp.float32),
        ],
        scratch_types=[
            pltpu.VMEM((b_per_w,), jnp.int32),
            pltpu.VMEM((b_per_w, E), jnp.float32),
            pltpu.VMEM((CB, HP), jnp.int32),
            pltpu.VMEM((CB, HP), jnp.int32),
            pltpu.VMEM((CB, HG, E), jnp.float32),
            pltpu.VMEM((CB, HG, E), jnp.float32),
            pltpu.VMEM((b_per_w, HP), jnp.float32),
            pltpu.VMEM((b_per_w * E,), jnp.float32),
            pltpu.SemaphoreType.DMA,
            pltpu.SemaphoreType.DMA,
            pltpu.SemaphoreType.DMA,
        ],
    )
    def gather_kernel(ut_hbm, it_hbm, uid_hbm, sid_hbm, hid_hbm, hw_hbm,
                      u_out, v_out, p_out,
                      idx_b, rows_b, idx_h0, idx_h1, rows_h0, rows_h1,
                      wv, pool_buf, sem_u, sem0, sem1):
        wid = lax.axis_index("s") * _NC + lax.axis_index("c")
        base = wid * b_per_w

        wcopy = pltpu.make_async_copy(
            hw_hbm.at[pl.ds(base, b_per_w)], wv, sem1)
        wcopy.start()

        pltpu.sync_copy(uid_hbm.at[pl.ds(base, b_per_w)], idx_b)
        pltpu.async_copy(ut_hbm.at[idx_b], rows_b, sem_u).wait()
        pltpu.sync_copy(rows_b, u_out.at[pl.ds(base, b_per_w)])
        pltpu.sync_copy(sid_hbm.at[pl.ds(base, b_per_w)], idx_b)
        pltpu.async_copy(it_hbm.at[idx_b], rows_b, sem_u).wait()
        pltpu.sync_copy(rows_b, v_out.at[pl.ds(base, b_per_w)])
        wcopy.wait()

        col = [lax.iota(jnp.int32, _L) + k * _L for k in range(E // _L)]

        def start_gather(c, idx_h, rows_h, sem):
            pltpu.sync_copy(hid_hbm.at[pl.ds(base + c * CB, CB)], idx_h)
            for b in range(CB):
                pltpu.make_async_copy(
                    it_hbm.at[idx_h.at[b, pl.ds(0, HG)]],
                    rows_h.at[b], sem).start()

        def compute_chunk(c, idx_h, rows_h, sem):
            for b in range(CB):
                pltpu.make_async_copy(
                    it_hbm.at[idx_h.at[b, pl.ds(0, HG)]],
                    rows_h.at[b], sem).wait()

            @pl.loop(0, CB)
            def _(b):
                bvec_l = jnp.full((_L,), b, dtype=jnp.int32)
                bvec_w = jnp.full((_L,), c * CB + b, dtype=jnp.int32)

                def jstep(j, acc):
                    jvec = jnp.full((_L,), j, dtype=jnp.int32)
                    wvec = plsc.load_gather(wv, [bvec_w, jvec])
                    return tuple(
                        acc[k] + wvec * plsc.load_gather(
                            rows_h, [bvec_l, jvec, col[k]])
                        for k in range(E // _L))

                acc = lax.fori_loop(
                    0, H, jstep,
                    tuple(jnp.zeros((_L,), jnp.float32)
                          for _ in range(E // _L)))
                pbase = (c * CB + b) * E
                for k in range(E // _L):
                    pool_buf[pl.ds(pbase + k * _L, _L)] = acc[k]

        start_gather(0, idx_h0, rows_h0, sem0)

        @pl.loop(0, n_chunks // 2)
        def _(cc):
            c = cc * 2

            start_gather(c + 1, idx_h1, rows_h1, sem1)
            compute_chunk(c, idx_h0, rows_h0, sem0)

            @pl.when(c + 2 < n_chunks)
            def _():
                start_gather(c + 2, idx_h0, rows_h0, sem0)
            compute_chunk(c + 1, idx_h1, rows_h1, sem1)

        pltpu.sync_copy(pool_buf, p_out.at[pl.ds(base * E, b_per_w * E)])

    return gather_kernel(user_table, item_table, user_id, song_id,
                         hist_ids, hist_weights)


def _tc_mlp(u, v, pooled, hist_weights, W1, b1, W2, b2, W3, b3, W_out, b_out):
    B, E = u.shape
    H = hist_weights.shape[1]
    BS = 512
    grid = (B // BS,)

    def body(u_ref, v_ref, p_ref, w_ref,
             W1_ref, b1_ref, W2_ref, b2_ref, W3_ref, b3_ref,
             Wo_ref, bo_ref, out_ref):
        w = w_ref[...]
        wsum = jnp.sum(w, axis=1, keepdims=True) + 1e-8
        hist = p_ref[...] / wsum
        uu = u_ref[...]
        vv = v_ref[...]
        x = jnp.concatenate([uu, vv, hist], axis=1)
        x = jnp.maximum(jnp.dot(x, W1_ref[...],
                                preferred_element_type=jnp.float32)
                        + b1_ref[...][None, :], 0.0)
        x = jnp.maximum(jnp.dot(x, W2_ref[...],
                                preferred_element_type=jnp.float32)
                        + b2_ref[...][None, :], 0.0)
        x = jnp.maximum(jnp.dot(x, W3_ref[...],
                                preferred_element_type=jnp.float32)
                        + b3_ref[...][None, :], 0.0)
        y = jnp.concatenate([uu * vv, x], axis=1)
        out_ref[...] = (jnp.dot(y, Wo_ref[...],
                                preferred_element_type=jnp.float32)
                        + bo_ref[...][None, :])

    rep = lambda *shape: pl.BlockSpec(shape, lambda i: (0,) * len(shape))
    return pl.pallas_call(
        body,
        grid=grid,
        in_specs=[
            pl.BlockSpec((BS, E), lambda i: (i, 0)),
            pl.BlockSpec((BS, E), lambda i: (i, 0)),
            pl.BlockSpec((BS, E), lambda i: (i, 0)),
            pl.BlockSpec((BS, H), lambda i: (i, 0)),
            rep(*W1.shape), rep(*b1.shape),
            rep(*W2.shape), rep(*b2.shape),
            rep(*W3.shape), rep(*b3.shape),
            rep(*W_out.shape), rep(*b_out.shape),
        ],
        out_specs=pl.BlockSpec((BS, 1), lambda i: (i, 0)),
        out_shape=jax.ShapeDtypeStruct((B, 1), jnp.float32),
    )(u, v, pooled, hist_weights, W1, b1, W2, b2, W3, b3, W_out, b_out)


def kernel(user_id, song_id, hist_ids, hist_weights, user_table, item_table,
           W1, b1, W2, b2, W3, b3, W_out, b_out):
    B, H = hist_ids.shape
    E = user_table.shape[1]
    hid_p = jnp.pad(hist_ids, ((0, 0), (0, 128 - H)))
    hw_p = jnp.pad(hist_weights, ((0, 0), (0, 128 - H)))
    u, v, pooled_flat = _sc_gather_pool(user_table, item_table,
                                        user_id, song_id, hid_p, hw_p)
    pooled = pooled_flat.reshape(B, E)
    return _tc_mlp(u, v, pooled, hist_weights,
                   W1, b1, W2, b2, W3, b3, W_out, b_out)


# speedup vs baseline: 2.7924x; 2.7924x over previous
import functools

import jax
import jax.numpy as jnp
from jax import lax
from jax.experimental import pallas as pl
from jax.experimental.pallas import tpu as pltpu
from jax.experimental.pallas import tpu_sc as plsc

_NC = 2
_NS = 16
_NW = _NC * _NS
_L = 16


def _pad_table(t):
    N, E = t.shape
    BR = 2000
    out = jax.ShapeDtypeStruct((N, 128), jnp.float32)

    def body(t_ref, o_ref):
        blk = t_ref[...]
        o_ref[...] = jnp.pad(blk, ((0, 0), (0, 128 - E)))

    return pl.pallas_call(
        body,
        grid=(N // BR,),
        in_specs=[pl.BlockSpec((BR, E), lambda i: (i, 0))],
        out_specs=pl.BlockSpec((BR, 128), lambda i: (i, 0)),
        out_shape=out,
    )(t)


def _sc_hist(it_p, hist_flat, hw_flat, B):
    NH = hist_flat.shape[0]
    EP = it_p.shape[1]
    E = 64
    H = NH // B
    b_per_w = B // _NW
    CB = 8
    CH = CB * H
    n_chunks = b_per_w // CB

    mesh = plsc.VectorSubcoreMesh(core_axis_name="c", subcore_axis_name="s")

    @functools.partial(
        pl.kernel,
        mesh=mesh,
        compiler_params=pltpu.CompilerParams(needs_layout_passes=False),
        out_type=jax.ShapeDtypeStruct((B * E,), jnp.float32),
        scratch_types=[
            pltpu.VMEM((CH,), jnp.int32),
            pltpu.VMEM((CH,), jnp.int32),
            pltpu.VMEM((CH, EP), jnp.float32),
            pltpu.VMEM((CH, EP), jnp.float32),
            pltpu.VMEM((b_per_w * H,), jnp.float32),
            pltpu.VMEM((b_per_w * E,), jnp.float32),
            pltpu.SemaphoreType.DMA,
            pltpu.SemaphoreType.DMA,
        ],
    )
    def hist_kernel(it_hbm, hid_hbm, hw_hbm, p_out,
                    idx_h0, idx_h1, rows_h0, rows_h1,
                    wv, pool_buf, sem0, sem1):
        wid = lax.axis_index("s") * _NC + lax.axis_index("c")
        base = wid * b_per_w
        hbase = base * H

        wcopy = pltpu.make_async_copy(
            hw_hbm.at[pl.ds(hbase, b_per_w * H)], wv, sem1)
        wcopy.start()

        col = [lax.iota(jnp.int32, _L) + k * _L for k in range(E // _L)]

        def start_gather(c, idx_h, rows_h, sem):
            off = hbase + c * CH
            pltpu.sync_copy(hid_hbm.at[pl.ds(off, CH)], idx_h)
            pltpu.make_async_copy(it_hbm.at[idx_h], rows_h, sem).start()

        def compute_chunk(c, idx_h, rows_h, sem):
            pltpu.make_async_copy(it_hbm.at[idx_h], rows_h, sem).wait()

            @pl.loop(0, CB)
            def _(b):
                rbase = b * H
                wbase = c * CH + rbase

                def jstep(j, acc):
                    rvec = jnp.full((_L,), rbase + j, dtype=jnp.int32)
                    wvec = plsc.load_gather(
                        wv, [jnp.full((_L,), wbase + j, dtype=jnp.int32)])
                    return tuple(
                        acc[k] + wvec * plsc.load_gather(rows_h, [rvec, col[k]])
                        for k in range(E // _L))

                acc = lax.fori_loop(
                    0, H, jstep,
                    tuple(jnp.zeros((_L,), jnp.float32)
                          for _ in range(E // _L)))
                pbase = (c * CB + b) * E
                for k in range(E // _L):
                    pool_buf[pl.ds(pbase + k * _L, _L)] = acc[k]

        start_gather(0, idx_h0, rows_h0, sem0)
        wcopy.wait()

        @pl.loop(0, n_chunks // 2)
        def _(cc):
            c = cc * 2
            start_gather(c + 1, idx_h1, rows_h1, sem1)
            compute_chunk(c, idx_h0, rows_h0, sem0)

            @pl.when(c + 2 < n_chunks)
            def _():
                start_gather(c + 2, idx_h0, rows_h0, sem0)
            compute_chunk(c + 1, idx_h1, rows_h1, sem1)

        pltpu.sync_copy(pool_buf, p_out.at[pl.ds(base * E, b_per_w * E)])

    return hist_kernel(it_p, hist_flat, hw_flat)


def _sc_uv(ut_p, it_p, user_id, song_id):
    B = user_id.shape[0]
    EP = ut_p.shape[1]
    b_per_w = B // _NW

    mesh = plsc.VectorSubcoreMesh(core_axis_name="c", subcore_axis_name="s")

    @functools.partial(
        pl.kernel,
        mesh=mesh,
        compiler_params=pltpu.CompilerParams(needs_layout_passes=False),
        out_type=[
            jax.ShapeDtypeStruct((B, EP), jnp.float32),
            jax.ShapeDtypeStruct((B, EP), jnp.float32),
        ],
        scratch_types=[
            pltpu.VMEM((b_per_w,), jnp.int32),
            pltpu.VMEM((b_per_w,), jnp.int32),
            pltpu.VMEM((b_per_w, EP), jnp.float32),
            pltpu.VMEM((b_per_w, EP), jnp.float32),
            pltpu.SemaphoreType.DMA,
            pltpu.SemaphoreType.DMA,
        ],
    )
    def uv_kernel(ut_hbm, it_hbm, uid_hbm, sid_hbm, u_out, v_out,
                  idx_u, idx_v, rows_u, rows_v, sem_u, sem_v):
        wid = lax.axis_index("s") * _NC + lax.axis_index("c")
        base = wid * b_per_w
        pltpu.sync_copy(uid_hbm.at[pl.ds(base, b_per_w)], idx_u)
        cu = pltpu.make_async_copy(ut_hbm.at[idx_u], rows_u, sem_u)
        cu.start()
        pltpu.sync_copy(sid_hbm.at[pl.ds(base, b_per_w)], idx_v)
        cv = pltpu.make_async_copy(it_hbm.at[idx_v], rows_v, sem_v)
        cv.start()
        cu.wait()
        pltpu.sync_copy(rows_u, u_out.at[pl.ds(base, b_per_w)])
        cv.wait()
        pltpu.sync_copy(rows_v, v_out.at[pl.ds(base, b_per_w)])

    return uv_kernel(ut_p, it_p, user_id, song_id)


def _tc_mlp(u, v, pooled, hist_weights, W1, b1, W2, b2, W3, b3, W_out, b_out):
    B, EP = u.shape
    E = pooled.shape[1]
    H = hist_weights.shape[1]
    BS = 512
    grid = (B // BS,)

    def body(u_ref, v_ref, p_ref, w_ref,
             W1_ref, b1_ref, W2_ref, b2_ref, W3_ref, b3_ref,
             Wo_ref, bo_ref, out_ref):
        w = w_ref[...]
        wsum = jnp.sum(w, axis=1, keepdims=True) + 1e-8
        hist = p_ref[...] / wsum
        uu = u_ref[...][:, :E]
        vv = v_ref[...][:, :E]
        x = jnp.concatenate([uu, vv, hist], axis=1)
        x = jnp.maximum(jnp.dot(x, W1_ref[...],
                                preferred_element_type=jnp.float32)
                        + b1_ref[...][None, :], 0.0)
        x = jnp.maximum(jnp.dot(x, W2_ref[...],
                                preferred_element_type=jnp.float32)
                        + b2_ref[...][None, :], 0.0)
        x = jnp.maximum(jnp.dot(x, W3_ref[...],
                                preferred_element_type=jnp.float32)
                        + b3_ref[...][None, :], 0.0)
        y = jnp.concatenate([uu * vv, x], axis=1)
        out_ref[...] = (jnp.dot(y, Wo_ref[...],
                                preferred_element_type=jnp.float32)
                        + bo_ref[...][None, :])

    rep = lambda *shape: pl.BlockSpec(shape, lambda i: (0,) * len(shape))
    return pl.pallas_call(
        body,
        grid=grid,
        in_specs=[
            pl.BlockSpec((BS, EP), lambda i: (i, 0)),
            pl.BlockSpec((BS, EP), lambda i: (i, 0)),
            pl.BlockSpec((BS, E), lambda i: (i, 0)),
            pl.BlockSpec((BS, H), lambda i: (i, 0)),
            rep(*W1.shape), rep(*b1.shape),
            rep(*W2.shape), rep(*b2.shape),
            rep(*W3.shape), rep(*b3.shape),
            rep(*W_out.shape), rep(*b_out.shape),
        ],
        out_specs=pl.BlockSpec((BS, 1), lambda i: (i, 0)),
        out_shape=jax.ShapeDtypeStruct((B, 1), jnp.float32),
    )(u, v, pooled, hist_weights, W1, b1, W2, b2, W3, b3, W_out, b_out)


def kernel(user_id, song_id, hist_ids, hist_weights, user_table, item_table,
           W1, b1, W2, b2, W3, b3, W_out, b_out):
    B, H = hist_ids.shape
    N, E = user_table.shape
    it_p = _pad_table(item_table)
    ut_p = _pad_table(user_table)
    hist_flat = hist_ids.reshape(-1)
    hw_flat = hist_weights.reshape(-1)
    pooled_flat = _sc_hist(it_p, hist_flat, hw_flat, B)
    u, v = _sc_uv(ut_p, it_p, user_id, song_id)
    pooled = pooled_flat.reshape(B, E)
    return _tc_mlp(u, v, pooled, hist_weights,
                   W1, b1, W2, b2, W3, b3, W_out, b_out)


# speedup vs baseline: 4.3298x; 1.5506x over previous
import functools

import jax
import jax.numpy as jnp
from jax import lax
from jax.experimental import pallas as pl
from jax.experimental.pallas import tpu as pltpu
from jax.experimental.pallas import tpu_sc as plsc

_NC = 2
_NS = 16
_NW = _NC * _NS
_L = 16


def _sc_hist_item(item_table, song_id, hist_flat, hw_flat):
    B = song_id.shape[0]
    NH = hist_flat.shape[0]
    E = item_table.shape[1]
    H = NH // B
    b_per_w = B // _NW
    CB = 8
    CH = CB * H
    n_chunks = b_per_w // CB

    mesh = plsc.VectorSubcoreMesh(core_axis_name="c", subcore_axis_name="s")

    @functools.partial(
        pl.kernel,
        mesh=mesh,
        compiler_params=pltpu.CompilerParams(use_tc_tiling_on_sc=False,
                                             needs_layout_passes=False),
        out_type=[
            jax.ShapeDtypeStruct((B, E), jnp.float32),
            jax.ShapeDtypeStruct((B * E,), jnp.float32),
        ],
        scratch_types=[
            pltpu.VMEM((b_per_w,), jnp.int32),
            pltpu.VMEM((b_per_w, E), jnp.float32),
            pltpu.VMEM((CH,), jnp.int32),
            pltpu.VMEM((CH,), jnp.int32),
            pltpu.VMEM((CH, E), jnp.float32),
            pltpu.VMEM((CH, E), jnp.float32),
            pltpu.VMEM((b_per_w * H,), jnp.float32),
            pltpu.VMEM((b_per_w * E,), jnp.float32),
            pltpu.SemaphoreType.DMA,
            pltpu.SemaphoreType.DMA,
            pltpu.SemaphoreType.DMA,
        ],
    )
    def hist_kernel(it_hbm, sid_hbm, hid_hbm, hw_hbm,
                    v_out, p_out,
                    idx_b, rows_b, idx_h0, idx_h1, rows_h0, rows_h1,
                    wv, pool_buf, sem_v, sem0, sem1):
        wid = lax.axis_index("s") * _NC + lax.axis_index("c")
        base = wid * b_per_w
        hbase = base * H

        wcopy = pltpu.make_async_copy(
            hw_hbm.at[pl.ds(hbase, b_per_w * H)], wv, sem1)
        wcopy.start()

        pltpu.sync_copy(sid_hbm.at[pl.ds(base, b_per_w)], idx_b)
        pltpu.async_copy(it_hbm.at[idx_b], rows_b, sem_v).wait()
        pltpu.sync_copy(rows_b, v_out.at[pl.ds(base, b_per_w)])
        wcopy.wait()

        col = [lax.iota(jnp.int32, _L) + k * _L for k in range(E // _L)]

        def start_gather(c, idx_h, rows_h, sem):
            off = hbase + c * CH
            pltpu.sync_copy(hid_hbm.at[pl.ds(off, CH)], idx_h)
            pltpu.make_async_copy(it_hbm.at[idx_h], rows_h, sem).start()

        def compute_chunk(c, idx_h, rows_h, sem):
            pltpu.make_async_copy(it_hbm.at[idx_h], rows_h, sem).wait()

            @pl.loop(0, CB)
            def _(b):
                rbase = b * H
                wbase = c * CH + rbase

                def jstep(j, acc):
                    rvec = jnp.full((_L,), rbase + j, dtype=jnp.int32)
                    wvec = plsc.load_gather(
                        wv, [jnp.full((_L,), wbase + j, dtype=jnp.int32)])
                    return tuple(
                        acc[k] + wvec * plsc.load_gather(rows_h, [rvec, col[k]])
                        for k in range(E // _L))

                acc = lax.fori_loop(
                    0, H, jstep,
                    tuple(jnp.zeros((_L,), jnp.float32)
                          for _ in range(E // _L)))
                pbase = (c * CB + b) * E
                for k in range(E // _L):
                    pool_buf[pl.ds(pbase + k * _L, _L)] = acc[k]

        start_gather(0, idx_h0, rows_h0, sem0)

        @pl.loop(0, n_chunks // 2)
        def _(cc):
            c = cc * 2
            start_gather(c + 1, idx_h1, rows_h1, sem1)
            compute_chunk(c, idx_h0, rows_h0, sem0)

            @pl.when(c + 2 < n_chunks)
            def _():
                start_gather(c + 2, idx_h0, rows_h0, sem0)
            compute_chunk(c + 1, idx_h1, rows_h1, sem1)

        pltpu.sync_copy(pool_buf, p_out.at[pl.ds(base * E, b_per_w * E)])

    return hist_kernel(item_table, song_id, hist_flat, hw_flat)


def _sc_ucols(ut_t, user_id):
    E, N = ut_t.shape
    B = user_id.shape[0]
    CPW = E // _NW

    mesh = plsc.VectorSubcoreMesh(core_axis_name="c", subcore_axis_name="s")

    @functools.partial(
        pl.kernel,
        mesh=mesh,
        compiler_params=pltpu.CompilerParams(needs_layout_passes=False),
        out_type=jax.ShapeDtypeStruct((E, B), jnp.float32),
        scratch_types=[
            pltpu.VMEM((N,), jnp.float32),
            pltpu.VMEM((B,), jnp.int32),
            pltpu.VMEM((B,), jnp.float32),
        ],
    )
    def ucol_kernel(ut_hbm, uid_hbm, u_out, col_buf, uid_v, acc_v):
        wid = lax.axis_index("s") * _NC + lax.axis_index("c")
        pltpu.sync_copy(uid_hbm, uid_v)

        @pl.loop(0, CPW)
        def _(cc):
            c = wid * CPW + cc
            pltpu.sync_copy(ut_hbm.at[c], col_buf)

            @pl.loop(0, B // _L)
            def _(g):
                idvec = uid_v[pl.ds(g * _L, _L)]
                acc_v[pl.ds(g * _L, _L)] = plsc.load_gather(col_buf, [idvec])
            pltpu.sync_copy(acc_v, u_out.at[c])

    return ucol_kernel(ut_t, user_id)


def _tc_mlp(u_t, v, pooled, w_t, W1, b1, W2, b2, W3, b3, W_out, b_out):
    E, B = u_t.shape
    H = w_t.shape[0]
    BS = 512
    grid = (B // BS,)

    def body(u_ref, v_ref, p_ref, w_ref,
             W1_ref, b1_ref, W2_ref, b2_ref, W3_ref, b3_ref,
             Wo_ref, bo_ref, out_ref):
        wsum = jnp.sum(w_ref[...], axis=0)[:, None] + 1e-8
        hist = p_ref[...] / wsum
        uu = u_ref[...].T
        vv = v_ref[...]
        x = jnp.concatenate([uu, vv, hist], axis=1)
        x = jnp.maximum(jnp.dot(x, W1_ref[...],
                                preferred_element_type=jnp.float32)
                        + b1_ref[...][None, :], 0.0)
        x = jnp.maximum(jnp.dot(x, W2_ref[...],
                                preferred_element_type=jnp.float32)
                        + b2_ref[...][None, :], 0.0)
        x = jnp.maximum(jnp.dot(x, W3_ref[...],
                                preferred_element_type=jnp.float32)
                        + b3_ref[...][None, :], 0.0)
        y = jnp.concatenate([uu * vv, x], axis=1)
        out_ref[...] = (jnp.dot(y, Wo_ref[...],
                                preferred_element_type=jnp.float32)
                        + bo_ref[...][None, :])

    rep = lambda *shape: pl.BlockSpec(shape, lambda i: (0,) * len(shape))
    return pl.pallas_call(
        body,
        grid=grid,
        in_specs=[
            pl.BlockSpec((E, BS), lambda i: (0, i)),
            pl.BlockSpec((BS, E), lambda i: (i, 0)),
            pl.BlockSpec((BS, E), lambda i: (i, 0)),
            pl.BlockSpec((H, BS), lambda i: (0, i)),
            rep(*W1.shape), rep(*b1.shape),
            rep(*W2.shape), rep(*b2.shape),
            rep(*W3.shape), rep(*b3.shape),
            rep(*W_out.shape), rep(*b_out.shape),
        ],
        out_specs=pl.BlockSpec((BS, 1), lambda i: (i, 0)),
        out_shape=jax.ShapeDtypeStruct((B, 1), jnp.float32),
    )(u_t, v, pooled, w_t, W1, b1, W2, b2, W3, b3, W_out, b_out)


def kernel(user_id, song_id, hist_ids, hist_weights, user_table, item_table,
           W1, b1, W2, b2, W3, b3, W_out, b_out):
    B, H = hist_ids.shape
    N, E = user_table.shape
    hist_flat = hist_ids.reshape(-1)
    hw_flat = hist_weights.reshape(-1)
    ut_t = user_table.T
    w_t = hist_weights.T
    u_t = _sc_ucols(ut_t, user_id)
    v, pooled_flat = _sc_hist_item(item_table, song_id, hist_flat, hw_flat)
    pooled = pooled_flat.reshape(B, E)
    return _tc_mlp(u_t, v, pooled, w_t,
                   W1, b1, W2, b2, W3, b3, W_out, b_out)


# speedup vs baseline: 4.6667x; 1.0778x over previous
import functools

import jax
import jax.numpy as jnp
from jax import lax
from jax.experimental import pallas as pl
from jax.experimental.pallas import tpu as pltpu
from jax.experimental.pallas import tpu_sc as plsc

_NC = 2
_NS = 16
_NW = _NC * _NS
_L = 16


def _sc_hist_item(item_table, song_id, hist_flat, hw_flat):
    B = song_id.shape[0]
    NH = hist_flat.shape[0]
    E = item_table.shape[1]
    H = NH // B
    b_per_w = B // _NW
    CB = 8
    CH = CB * H
    n_chunks = b_per_w // CB

    mesh = plsc.VectorSubcoreMesh(core_axis_name="c", subcore_axis_name="s")

    @functools.partial(
        pl.kernel,
        mesh=mesh,
        compiler_params=pltpu.CompilerParams(use_tc_tiling_on_sc=False,
                                             needs_layout_passes=False),
        out_type=[
            jax.ShapeDtypeStruct((B, E), jnp.float32),
            jax.ShapeDtypeStruct((B * E,), jnp.float32),
        ],
        scratch_types=[
            pltpu.VMEM((b_per_w,), jnp.int32),
            pltpu.VMEM((b_per_w, E), jnp.float32),
            pltpu.VMEM((CH,), jnp.int32),
            pltpu.VMEM((CH,), jnp.int32),
            pltpu.VMEM((CH, E), jnp.float32),
            pltpu.VMEM((CH, E), jnp.float32),
            pltpu.VMEM((b_per_w * H,), jnp.float32),
            pltpu.VMEM((b_per_w * E,), jnp.float32),
            pltpu.SemaphoreType.DMA,
            pltpu.SemaphoreType.DMA,
            pltpu.SemaphoreType.DMA,
        ],
    )
    def hist_kernel(it_hbm, sid_hbm, hid_hbm, hw_hbm,
                    v_out, p_out,
                    idx_b, rows_b, idx_h0, idx_h1, rows_h0, rows_h1,
                    wv, pool_buf, sem_v, sem0, sem1):
        wid = lax.axis_index("s") * _NC + lax.axis_index("c")
        base = wid * b_per_w
        hbase = base * H

        wcopy = pltpu.make_async_copy(
            hw_hbm.at[pl.ds(hbase, b_per_w * H)], wv, sem1)
        wcopy.start()

        pltpu.sync_copy(sid_hbm.at[pl.ds(base, b_per_w)], idx_b)
        pltpu.async_copy(it_hbm.at[idx_b], rows_b, sem_v).wait()
        pltpu.sync_copy(rows_b, v_out.at[pl.ds(base, b_per_w)])
        wcopy.wait()

        col = [lax.iota(jnp.int32, _L) + k * _L for k in range(E // _L)]

        def start_gather(c, idx_h, rows_h, sem):
            off = hbase + c * CH
            pltpu.sync_copy(hid_hbm.at[pl.ds(off, CH)], idx_h)
            pltpu.make_async_copy(it_hbm.at[idx_h], rows_h, sem).start()

        def compute_chunk(c, idx_h, rows_h, sem):
            pltpu.make_async_copy(it_hbm.at[idx_h], rows_h, sem).wait()

            @pl.loop(0, CB)
            def _(b):
                rbase = b * H
                wbase = c * CH + rbase

                def jstep(j, acc):
                    rvec = jnp.full((_L,), rbase + j, dtype=jnp.int32)
                    wvec = plsc.load_gather(
                        wv, [jnp.full((_L,), wbase + j, dtype=jnp.int32)])
                    return tuple(
                        acc[k] + wvec * plsc.load_gather(rows_h, [rvec, col[k]])
                        for k in range(E // _L))

                acc = lax.fori_loop(
                    0, H, jstep,
                    tuple(jnp.zeros((_L,), jnp.float32)
                          for _ in range(E // _L)))
                pbase = (c * CB + b) * E
                for k in range(E // _L):
                    pool_buf[pl.ds(pbase + k * _L, _L)] = acc[k]

        start_gather(0, idx_h0, rows_h0, sem0)

        @pl.loop(0, n_chunks // 2)
        def _(cc):
            c = cc * 2
            start_gather(c + 1, idx_h1, rows_h1, sem1)
            compute_chunk(c, idx_h0, rows_h0, sem0)

            @pl.when(c + 2 < n_chunks)
            def _():
                start_gather(c + 2, idx_h0, rows_h0, sem0)
            compute_chunk(c + 1, idx_h1, rows_h1, sem1)

        pltpu.sync_copy(pool_buf, p_out.at[pl.ds(base * E, b_per_w * E)])

    return hist_kernel(item_table, song_id, hist_flat, hw_flat)


def _sc_ucols(ut_t, user_id):
    E, N = ut_t.shape
    B = user_id.shape[0]
    CPW = E // _NW

    mesh = plsc.VectorSubcoreMesh(core_axis_name="c", subcore_axis_name="s")

    @functools.partial(
        pl.kernel,
        mesh=mesh,
        compiler_params=pltpu.CompilerParams(needs_layout_passes=False),
        out_type=jax.ShapeDtypeStruct((E, B), jnp.float32),
        scratch_types=[
            pltpu.VMEM((N,), jnp.float32),
            pltpu.VMEM((B,), jnp.int32),
            pltpu.VMEM((B,), jnp.float32),
        ],
    )
    def ucol_kernel(ut_hbm, uid_hbm, u_out, col_buf, uid_v, acc_v):
        wid = lax.axis_index("s") * _NC + lax.axis_index("c")
        pltpu.sync_copy(uid_hbm, uid_v)

        @pl.loop(0, CPW)
        def _(cc):
            c = wid * CPW + cc
            pltpu.sync_copy(ut_hbm.at[c], col_buf)

            @pl.loop(0, B // _L)
            def _(g):
                idvec = uid_v[pl.ds(g * _L, _L)]
                acc_v[pl.ds(g * _L, _L)] = plsc.load_gather(col_buf, [idvec])
            pltpu.sync_copy(acc_v, u_out.at[c])

    return ucol_kernel(ut_t, user_id)


def _tc_mlp(u_t, v, pooled, w_t, W1, b1, W2, b2, W3, b3, W_out, b_out):
    E, B = u_t.shape
    H = w_t.shape[0]
    BS = 512
    grid = (B // BS,)

    def body(u_ref, v_ref, p_ref, w_ref,
             W1_ref, b1_ref, W2_ref, b2_ref, W3_ref, b3_ref,
             Wo_ref, bo_ref, out_ref):
        wsum = jnp.sum(w_ref[...], axis=0)[:, None] + 1e-8
        hist = p_ref[...] / wsum
        uu = u_ref[...].T
        vv = v_ref[...]
        x = jnp.concatenate([uu, vv, hist], axis=1)
        x = jnp.maximum(jnp.dot(x, W1_ref[...],
                                preferred_element_type=jnp.float32)
                        + b1_ref[...][None, :], 0.0)
        x = jnp.maximum(jnp.dot(x, W2_ref[...],
                                preferred_element_type=jnp.float32)
                        + b2_ref[...][None, :], 0.0)
        x = jnp.maximum(jnp.dot(x, W3_ref[...],
                                preferred_element_type=jnp.float32)
                        + b3_ref[...][None, :], 0.0)
        y = jnp.concatenate([uu * vv, x], axis=1)
        out_ref[...] = (jnp.dot(y, Wo_ref[...],
                                preferred_element_type=jnp.float32)
                        + bo_ref[...][None, :])

    rep = lambda *shape: pl.BlockSpec(shape, lambda i: (0,) * len(shape))
    return pl.pallas_call(
        body,
        grid=grid,
        in_specs=[
            pl.BlockSpec((E, BS), lambda i: (0, i)),
            pl.BlockSpec((BS, E), lambda i: (i, 0)),
            pl.BlockSpec((BS, E), lambda i: (i, 0)),
            pl.BlockSpec((H, BS), lambda i: (0, i)),
            rep(*W1.shape), rep(*b1.shape),
            rep(*W2.shape), rep(*b2.shape),
            rep(*W3.shape), rep(*b3.shape),
            rep(*W_out.shape), rep(*b_out.shape),
        ],
        out_specs=pl.BlockSpec((BS, 1), lambda i: (i, 0)),
        out_shape=jax.ShapeDtypeStruct((B, 1), jnp.float32),
    )(u_t, v, pooled, w_t, W1, b1, W2, b2, W3, b3, W_out, b_out)


def kernel(user_id, song_id, hist_ids, hist_weights, user_table, item_table,
           W1, b1, W2, b2, W3, b3, W_out, b_out):
    B, H = hist_ids.shape
    N, E = user_table.shape
    hist_flat = hist_ids.reshape(-1)
    hw_flat = hist_weights.reshape(-1)
    ut_t = user_table.T
    w_t = hist_weights.T
    u_t = _sc_ucols(ut_t, user_id)
    song_id, u_t = lax.optimization_barrier((song_id, u_t))
    v, pooled_flat = _sc_hist_item(item_table, song_id, hist_flat, hw_flat)
    pooled = pooled_flat.reshape(B, E)
    return _tc_mlp(u_t, v, pooled, w_t,
                   W1, b1, W2, b2, W3, b3, W_out, b_out)


# speedup vs baseline: 4.7220x; 1.0118x over previous
import functools

import jax
import jax.numpy as jnp
from jax import lax
from jax.experimental import pallas as pl
from jax.experimental.pallas import tpu as pltpu
from jax.experimental.pallas import tpu_sc as plsc

_NC = 2
_NS = 16
_NW = _NC * _NS
_L = 16


def _sc_hist_item(item_table, hist_flat, hw_flat, B):
    NH = hist_flat.shape[0]
    E = item_table.shape[1]
    H = NH // B
    b_per_w = B // _NW
    CB = 8
    CH = CB * H
    n_chunks = b_per_w // CB

    mesh = plsc.VectorSubcoreMesh(core_axis_name="c", subcore_axis_name="s")

    @functools.partial(
        pl.kernel,
        mesh=mesh,
        compiler_params=pltpu.CompilerParams(use_tc_tiling_on_sc=False,
                                             needs_layout_passes=False),
        out_type=jax.ShapeDtypeStruct((B * E,), jnp.float32),
        scratch_types=[
            pltpu.VMEM((CH,), jnp.int32),
            pltpu.VMEM((CH,), jnp.int32),
            pltpu.VMEM((CH, E), jnp.float32),
            pltpu.VMEM((CH, E), jnp.float32),
            pltpu.VMEM((b_per_w * H,), jnp.float32),
            pltpu.VMEM((b_per_w * E,), jnp.float32),
            pltpu.SemaphoreType.DMA,
            pltpu.SemaphoreType.DMA,
        ],
    )
    def hist_kernel(it_hbm, hid_hbm, hw_hbm, p_out,
                    idx_h0, idx_h1, rows_h0, rows_h1,
                    wv, pool_buf, sem0, sem1):
        wid = lax.axis_index("s") * _NC + lax.axis_index("c")
        base = wid * b_per_w
        hbase = base * H

        wcopy = pltpu.make_async_copy(
            hw_hbm.at[pl.ds(hbase, b_per_w * H)], wv, sem1)
        wcopy.start()

        col = [lax.iota(jnp.int32, _L) + k * _L for k in range(E // _L)]

        def start_gather(c, idx_h, rows_h, sem):
            off = hbase + c * CH
            pltpu.sync_copy(hid_hbm.at[pl.ds(off, CH)], idx_h)
            pltpu.make_async_copy(it_hbm.at[idx_h], rows_h, sem).start()

        def compute_chunk(c, idx_h, rows_h, sem):
            pltpu.make_async_copy(it_hbm.at[idx_h], rows_h, sem).wait()

            @pl.loop(0, CB)
            def _(b):
                rbase = b * H
                wbase = c * CH + rbase

                def jstep(j, acc):
                    rvec = jnp.full((_L,), rbase + j, dtype=jnp.int32)
                    wvec = plsc.load_gather(
                        wv, [jnp.full((_L,), wbase + j, dtype=jnp.int32)])
                    return tuple(
                        acc[k] + wvec * plsc.load_gather(rows_h, [rvec, col[k]])
                        for k in range(E // _L))

                acc = lax.fori_loop(
                    0, H, jstep,
                    tuple(jnp.zeros((_L,), jnp.float32)
                          for _ in range(E // _L)))
                pbase = (c * CB + b) * E
                for k in range(E // _L):
                    pool_buf[pl.ds(pbase + k * _L, _L)] = acc[k]

        start_gather(0, idx_h0, rows_h0, sem0)
        wcopy.wait()

        @pl.loop(0, n_chunks // 2)
        def _(cc):
            c = cc * 2
            start_gather(c + 1, idx_h1, rows_h1, sem1)
            compute_chunk(c, idx_h0, rows_h0, sem0)

            @pl.when(c + 2 < n_chunks)
            def _():
                start_gather(c + 2, idx_h0, rows_h0, sem0)
            compute_chunk(c + 1, idx_h1, rows_h1, sem1)

        pltpu.sync_copy(pool_buf, p_out.at[pl.ds(base * E, b_per_w * E)])

    return hist_kernel(item_table, hist_flat, hw_flat)


def _sc_uvcols(ut_t, it_t, user_id, song_id):
    E, N = ut_t.shape
    B = user_id.shape[0]
    CPW = E // _NW

    mesh = plsc.VectorSubcoreMesh(core_axis_name="c", subcore_axis_name="s")

    @functools.partial(
        pl.kernel,
        mesh=mesh,
        compiler_params=pltpu.CompilerParams(needs_layout_passes=False),
        out_type=[
            jax.ShapeDtypeStruct((E, B), jnp.float32),
            jax.ShapeDtypeStruct((E, B), jnp.float32),
        ],
        scratch_types=[
            pltpu.VMEM((N,), jnp.float32),
            pltpu.VMEM((B,), jnp.int32),
            pltpu.VMEM((B,), jnp.int32),
            pltpu.VMEM((B,), jnp.float32),
        ],
    )
    def uvcol_kernel(ut_hbm, it_hbm, uid_hbm, sid_hbm, u_out, v_out,
                     col_buf, uid_v, sid_v, acc_v):
        wid = lax.axis_index("s") * _NC + lax.axis_index("c")
        pltpu.sync_copy(uid_hbm, uid_v)
        pltpu.sync_copy(sid_hbm, sid_v)

        def one_col(tab_hbm, ids_v, out_ref, c):
            pltpu.sync_copy(tab_hbm.at[c], col_buf)

            @pl.loop(0, B // _L)
            def _(g):
                idvec = ids_v[pl.ds(g * _L, _L)]
                acc_v[pl.ds(g * _L, _L)] = plsc.load_gather(col_buf, [idvec])
            pltpu.sync_copy(acc_v, out_ref.at[c])

        @pl.loop(0, CPW)
        def _(cc):
            c = wid * CPW + cc
            one_col(it_hbm, sid_v, v_out, c)
            one_col(ut_hbm, uid_v, u_out, c)

    return uvcol_kernel(ut_t, it_t, user_id, song_id)


def _tc_mlp(u_t, v_t, pooled, w_t, W1, b1, W2, b2, W3, b3, W_out, b_out):
    E, B = u_t.shape
    H = w_t.shape[0]
    BS = 1024
    grid = (B // BS,)

    def body(u_ref, v_ref, p_ref, w_ref,
             W1_ref, b1_ref, W2_ref, b2_ref, W3_ref, b3_ref,
             Wo_ref, bo_ref, out_ref):
        wsum = jnp.sum(w_ref[...], axis=0)[:, None] + 1e-8
        hist = p_ref[...] / wsum
        uu = u_ref[...].T
        vv = v_ref[...].T
        x = jnp.concatenate([uu, vv, hist], axis=1)
        x = jnp.maximum(jnp.dot(x, W1_ref[...],
                                preferred_element_type=jnp.float32)
                        + b1_ref[...][None, :], 0.0)
        x = jnp.maximum(jnp.dot(x, W2_ref[...],
                                preferred_element_type=jnp.float32)
                        + b2_ref[...][None, :], 0.0)
        x = jnp.maximum(jnp.dot(x, W3_ref[...],
                                preferred_element_type=jnp.float32)
                        + b3_ref[...][None, :], 0.0)
        y = jnp.concatenate([uu * vv, x], axis=1)
        out_ref[...] = (jnp.dot(y, Wo_ref[...],
                                preferred_element_type=jnp.float32)
                        + bo_ref[...][None, :])

    rep = lambda *shape: pl.BlockSpec(shape, lambda i: (0,) * len(shape))
    return pl.pallas_call(
        body,
        grid=grid,
        in_specs=[
            pl.BlockSpec((E, BS), lambda i: (0, i)),
            pl.BlockSpec((E, BS), lambda i: (0, i)),
            pl.BlockSpec((BS, E), lambda i: (i, 0)),
            pl.BlockSpec((H, BS), lambda i: (0, i)),
            rep(*W1.shape), rep(*b1.shape),
            rep(*W2.shape), rep(*b2.shape),
            rep(*W3.shape), rep(*b3.shape),
            rep(*W_out.shape), rep(*b_out.shape),
        ],
        out_specs=pl.BlockSpec((BS, 1), lambda i: (i, 0)),
        out_shape=jax.ShapeDtypeStruct((B, 1), jnp.float32),
    )(u_t, v_t, pooled, w_t, W1, b1, W2, b2, W3, b3, W_out, b_out)


def kernel(user_id, song_id, hist_ids, hist_weights, user_table, item_table,
           W1, b1, W2, b2, W3, b3, W_out, b_out):
    B, H = hist_ids.shape
    N, E = user_table.shape
    hist_flat = hist_ids.reshape(-1)
    hw_flat = hist_weights.reshape(-1)
    ut_t = user_table.T
    it_t = item_table.T
    w_t = hist_weights.T
    u_t, v_t = _sc_uvcols(ut_t, it_t, user_id, song_id)
    hist_flat, u_t = lax.optimization_barrier((hist_flat, u_t))
    pooled_flat = _sc_hist_item(item_table, hist_flat, hw_flat, B)
    pooled = pooled_flat.reshape(B, E)
    return _tc_mlp(u_t, v_t, pooled, w_t,
                   W1, b1, W2, b2, W3, b3, W_out, b_out)


# speedup vs baseline: 4.7886x; 1.0141x over previous
import functools

import jax
import jax.numpy as jnp
from jax import lax
from jax.experimental import pallas as pl
from jax.experimental.pallas import tpu as pltpu
from jax.experimental.pallas import tpu_sc as plsc

_NC = 2
_NS = 16
_NW = _NC * _NS
_L = 16


def _sc_hist_item(item_table, hist_flat, hw_flat, B):
    NH = hist_flat.shape[0]
    E = item_table.shape[1]
    H = NH // B
    b_per_w = B // _NW
    CB = 16
    CH = CB * H
    n_chunks = b_per_w // CB

    mesh = plsc.VectorSubcoreMesh(core_axis_name="c", subcore_axis_name="s")

    @functools.partial(
        pl.kernel,
        mesh=mesh,
        compiler_params=pltpu.CompilerParams(use_tc_tiling_on_sc=False,
                                             needs_layout_passes=False),
        out_type=jax.ShapeDtypeStruct((B * E,), jnp.float32),
        scratch_types=[
            pltpu.VMEM((CH,), jnp.int32),
            pltpu.VMEM((CH,), jnp.int32),
            pltpu.VMEM((CH, E), jnp.float32),
            pltpu.VMEM((CH, E), jnp.float32),
            pltpu.VMEM((b_per_w * H,), jnp.float32),
            pltpu.VMEM((b_per_w * E,), jnp.float32),
            pltpu.SemaphoreType.DMA,
            pltpu.SemaphoreType.DMA,
        ],
    )
    def hist_kernel(it_hbm, hid_hbm, hw_hbm, p_out,
                    idx_h0, idx_h1, rows_h0, rows_h1,
                    wv, pool_buf, sem0, sem1):
        wid = lax.axis_index("s") * _NC + lax.axis_index("c")
        base = wid * b_per_w
        hbase = base * H

        wcopy = pltpu.make_async_copy(
            hw_hbm.at[pl.ds(hbase, b_per_w * H)], wv, sem1)
        wcopy.start()

        col = [lax.iota(jnp.int32, _L) + k * _L for k in range(E // _L)]

        def start_gather(c, idx_h, rows_h, sem):
            off = hbase + c * CH
            pltpu.sync_copy(hid_hbm.at[pl.ds(off, CH)], idx_h)
            pltpu.make_async_copy(it_hbm.at[idx_h], rows_h, sem).start()

        def compute_chunk(c, idx_h, rows_h, sem):
            pltpu.make_async_copy(it_hbm.at[idx_h], rows_h, sem).wait()

            @pl.loop(0, CB)
            def _(b):
                rbase = b * H
                wbase = c * CH + rbase

                def jstep(j, acc):
                    rvec = jnp.full((_L,), rbase + j, dtype=jnp.int32)
                    wvec = plsc.load_gather(
                        wv, [jnp.full((_L,), wbase + j, dtype=jnp.int32)])
                    return tuple(
                        acc[k] + wvec * plsc.load_gather(rows_h, [rvec, col[k]])
                        for k in range(E // _L))

                acc = lax.fori_loop(
                    0, H, jstep,
                    tuple(jnp.zeros((_L,), jnp.float32)
                          for _ in range(E // _L)))
                pbase = (c * CB + b) * E
                for k in range(E // _L):
                    pool_buf[pl.ds(pbase + k * _L, _L)] = acc[k]

        start_gather(0, idx_h0, rows_h0, sem0)
        wcopy.wait()

        @pl.loop(0, n_chunks // 2)
        def _(cc):
            c = cc * 2
            start_gather(c + 1, idx_h1, rows_h1, sem1)
            compute_chunk(c, idx_h0, rows_h0, sem0)

            @pl.when(c + 2 < n_chunks)
            def _():
                start_gather(c + 2, idx_h0, rows_h0, sem0)
            compute_chunk(c + 1, idx_h1, rows_h1, sem1)

        pltpu.sync_copy(pool_buf, p_out.at[pl.ds(base * E, b_per_w * E)])

    return hist_kernel(item_table, hist_flat, hw_flat)


def _sc_uvcols(ut_t, it_t, user_id, song_id):
    E, N = ut_t.shape
    B = user_id.shape[0]
    CPW = E // _NW

    mesh = plsc.VectorSubcoreMesh(core_axis_name="c", subcore_axis_name="s")

    @functools.partial(
        pl.kernel,
        mesh=mesh,
        compiler_params=pltpu.CompilerParams(needs_layout_passes=False),
        out_type=[
            jax.ShapeDtypeStruct((E, B), jnp.float32),
            jax.ShapeDtypeStruct((E, B), jnp.float32),
        ],
        scratch_types=[
            pltpu.VMEM((N,), jnp.float32),
            pltpu.VMEM((B,), jnp.int32),
            pltpu.VMEM((B,), jnp.int32),
            pltpu.VMEM((B,), jnp.float32),
        ],
    )
    def uvcol_kernel(ut_hbm, it_hbm, uid_hbm, sid_hbm, u_out, v_out,
                     col_buf, uid_v, sid_v, acc_v):
        wid = lax.axis_index("s") * _NC + lax.axis_index("c")
        pltpu.sync_copy(uid_hbm, uid_v)
        pltpu.sync_copy(sid_hbm, sid_v)

        def one_col(tab_hbm, ids_v, out_ref, c):
            pltpu.sync_copy(tab_hbm.at[c], col_buf)

            @pl.loop(0, B // _L)
            def _(g):
                idvec = ids_v[pl.ds(g * _L, _L)]
                acc_v[pl.ds(g * _L, _L)] = plsc.load_gather(col_buf, [idvec])
            pltpu.sync_copy(acc_v, out_ref.at[c])

        @pl.loop(0, CPW)
        def _(cc):
            c = wid * CPW + cc
            one_col(it_hbm, sid_v, v_out, c)
            one_col(ut_hbm, uid_v, u_out, c)

    return uvcol_kernel(ut_t, it_t, user_id, song_id)


def _tc_mlp(u_t, v_t, pooled, w_t, W1, b1, W2, b2, W3, b3, W_out, b_out):
    E, B = u_t.shape
    H = w_t.shape[0]
    BS = 1024
    grid = (B // BS,)

    def body(u_ref, v_ref, p_ref, w_ref,
             W1_ref, b1_ref, W2_ref, b2_ref, W3_ref, b3_ref,
             Wo_ref, bo_ref, out_ref):
        wsum = jnp.sum(w_ref[...], axis=0)[:, None] + 1e-8
        hist = p_ref[...] / wsum
        uu = u_ref[...].T
        vv = v_ref[...].T
        x = jnp.concatenate([uu, vv, hist], axis=1)
        x = jnp.maximum(jnp.dot(x, W1_ref[...],
                                preferred_element_type=jnp.float32)
                        + b1_ref[...][None, :], 0.0)
        x = jnp.maximum(jnp.dot(x, W2_ref[...],
                                preferred_element_type=jnp.float32)
                        + b2_ref[...][None, :], 0.0)
        x = jnp.maximum(jnp.dot(x, W3_ref[...],
                                preferred_element_type=jnp.float32)
                        + b3_ref[...][None, :], 0.0)
        y = jnp.concatenate([uu * vv, x], axis=1)
        out_ref[...] = (jnp.dot(y, Wo_ref[...],
                                preferred_element_type=jnp.float32)
                        + bo_ref[...][None, :])

    rep = lambda *shape: pl.BlockSpec(shape, lambda i: (0,) * len(shape))
    return pl.pallas_call(
        body,
        grid=grid,
        in_specs=[
            pl.BlockSpec((E, BS), lambda i: (0, i)),
            pl.BlockSpec((E, BS), lambda i: (0, i)),
            pl.BlockSpec((BS, E), lambda i: (i, 0)),
            pl.BlockSpec((H, BS), lambda i: (0, i)),
            rep(*W1.shape), rep(*b1.shape),
            rep(*W2.shape), rep(*b2.shape),
            rep(*W3.shape), rep(*b3.shape),
            rep(*W_out.shape), rep(*b_out.shape),
        ],
        out_specs=pl.BlockSpec((BS, 1), lambda i: (i, 0)),
        out_shape=jax.ShapeDtypeStruct((B, 1), jnp.float32),
    )(u_t, v_t, pooled, w_t, W1, b1, W2, b2, W3, b3, W_out, b_out)


def kernel(user_id, song_id, hist_ids, hist_weights, user_table, item_table,
           W1, b1, W2, b2, W3, b3, W_out, b_out):
    B, H = hist_ids.shape
    N, E = user_table.shape
    hist_flat = hist_ids.reshape(-1)
    hw_flat = hist_weights.reshape(-1)
    ut_t = user_table.T
    it_t = item_table.T
    w_t = hist_weights.T
    u_t, v_t = _sc_uvcols(ut_t, it_t, user_id, song_id)
    hist_flat, u_t = lax.optimization_barrier((hist_flat, u_t))
    pooled_flat = _sc_hist_item(item_table, hist_flat, hw_flat, B)
    pooled = pooled_flat.reshape(B, E)
    return _tc_mlp(u_t, v_t, pooled, w_t,
                   W1, b1, W2, b2, W3, b3, W_out, b_out)


# speedup vs baseline: 4.8836x; 1.0198x over previous
import functools

import jax
import jax.numpy as jnp
from jax import lax
from jax.experimental import pallas as pl
from jax.experimental.pallas import tpu as pltpu
from jax.experimental.pallas import tpu_sc as plsc

_NC = 2
_NS = 16
_NW = _NC * _NS
_L = 16


def _sc_hist_item(item_table, hist_flat, hw_flat, B):
    NH = hist_flat.shape[0]
    E = item_table.shape[1]
    H = NH // B
    b_per_w = B // _NW
    CB = 16
    CH = CB * H
    n_chunks = b_per_w // CB

    mesh = plsc.VectorSubcoreMesh(core_axis_name="c", subcore_axis_name="s")

    @functools.partial(
        pl.kernel,
        mesh=mesh,
        compiler_params=pltpu.CompilerParams(use_tc_tiling_on_sc=False,
                                             needs_layout_passes=False),
        out_type=jax.ShapeDtypeStruct((B * E,), jnp.float32),
        scratch_types=[
            pltpu.VMEM((CH,), jnp.int32),
            pltpu.VMEM((CH,), jnp.int32),
            pltpu.VMEM((CH, E), jnp.float32),
            pltpu.VMEM((CH, E), jnp.float32),
            pltpu.VMEM((b_per_w * H,), jnp.float32),
            pltpu.VMEM((b_per_w * E,), jnp.float32),
            pltpu.SemaphoreType.DMA,
            pltpu.SemaphoreType.DMA,
        ],
    )
    def hist_kernel(it_hbm, hid_hbm, hw_hbm, p_out,
                    idx_h0, idx_h1, rows_h0, rows_h1,
                    wv, pool_buf, sem0, sem1):
        wid = lax.axis_index("s") * _NC + lax.axis_index("c")
        base = wid * b_per_w
        hbase = base * H

        wcopy = pltpu.make_async_copy(
            hw_hbm.at[pl.ds(hbase, b_per_w * H)], wv, sem1)
        wcopy.start()

        col = [lax.iota(jnp.int32, _L) + k * _L for k in range(E // _L)]

        def start_gather(c, idx_h, rows_h, sem):
            off = hbase + c * CH
            pltpu.sync_copy(hid_hbm.at[pl.ds(off, CH)], idx_h)
            pltpu.make_async_copy(it_hbm.at[idx_h], rows_h, sem).start()

        def compute_chunk(c, idx_h, rows_h, sem):
            pltpu.make_async_copy(it_hbm.at[idx_h], rows_h, sem).wait()

            @pl.loop(0, CB)
            def _(b):
                rbase = b * H
                wbase = c * CH + rbase

                def jstep(j, acc):
                    rvec = jnp.full((_L,), rbase + j, dtype=jnp.int32)
                    wvec = plsc.load_gather(
                        wv, [jnp.full((_L,), wbase + j, dtype=jnp.int32)])
                    return tuple(
                        acc[k] + wvec * plsc.load_gather(rows_h, [rvec, col[k]])
                        for k in range(E // _L))

                acc = lax.fori_loop(
                    0, H, jstep,
                    tuple(jnp.zeros((_L,), jnp.float32)
                          for _ in range(E // _L)),
                    unroll=2)
                pbase = (c * CB + b) * E
                for k in range(E // _L):
                    pool_buf[pl.ds(pbase + k * _L, _L)] = acc[k]

        start_gather(0, idx_h0, rows_h0, sem0)
        wcopy.wait()

        @pl.loop(0, n_chunks // 2)
        def _(cc):
            c = cc * 2
            start_gather(c + 1, idx_h1, rows_h1, sem1)
            compute_chunk(c, idx_h0, rows_h0, sem0)

            @pl.when(c + 2 < n_chunks)
            def _():
                start_gather(c + 2, idx_h0, rows_h0, sem0)
            compute_chunk(c + 1, idx_h1, rows_h1, sem1)

        pltpu.sync_copy(pool_buf, p_out.at[pl.ds(base * E, b_per_w * E)])

    return hist_kernel(item_table, hist_flat, hw_flat)


def _sc_uvcols(ut_t, it_t, user_id, song_id):
    E, N = ut_t.shape
    B = user_id.shape[0]
    CPW = E // _NW

    mesh = plsc.VectorSubcoreMesh(core_axis_name="c", subcore_axis_name="s")

    @functools.partial(
        pl.kernel,
        mesh=mesh,
        compiler_params=pltpu.CompilerParams(needs_layout_passes=False),
        out_type=[
            jax.ShapeDtypeStruct((E, B), jnp.float32),
            jax.ShapeDtypeStruct((E, B), jnp.float32),
        ],
        scratch_types=[
            pltpu.VMEM((N,), jnp.float32),
            pltpu.VMEM((B,), jnp.int32),
            pltpu.VMEM((B,), jnp.int32),
            pltpu.VMEM((B,), jnp.float32),
        ],
    )
    def uvcol_kernel(ut_hbm, it_hbm, uid_hbm, sid_hbm, u_out, v_out,
                     col_buf, uid_v, sid_v, acc_v):
        wid = lax.axis_index("s") * _NC + lax.axis_index("c")
        pltpu.sync_copy(uid_hbm, uid_v)
        pltpu.sync_copy(sid_hbm, sid_v)

        def one_col(tab_hbm, ids_v, out_ref, c):
            pltpu.sync_copy(tab_hbm.at[c], col_buf)

            @pl.loop(0, B // _L)
            def _(g):
                idvec = ids_v[pl.ds(g * _L, _L)]
                acc_v[pl.ds(g * _L, _L)] = plsc.load_gather(col_buf, [idvec])
            pltpu.sync_copy(acc_v, out_ref.at[c])

        @pl.loop(0, CPW)
        def _(cc):
            c = wid * CPW + cc
            one_col(it_hbm, sid_v, v_out, c)
            one_col(ut_hbm, uid_v, u_out, c)

    return uvcol_kernel(ut_t, it_t, user_id, song_id)


def _tc_mlp(u_t, v_t, pooled, w_t, W1, b1, W2, b2, W3, b3, W_out, b_out):
    E, B = u_t.shape
    H = w_t.shape[0]
    BS = 1024
    grid = (B // BS,)

    def body(u_ref, v_ref, p_ref, w_ref,
             W1_ref, b1_ref, W2_ref, b2_ref, W3_ref, b3_ref,
             Wo_ref, bo_ref, out_ref):
        wsum = jnp.sum(w_ref[...], axis=0)[:, None] + 1e-8
        hist = p_ref[...] / wsum
        uu = u_ref[...].T
        vv = v_ref[...].T
        x = jnp.concatenate([uu, vv, hist], axis=1)
        x = jnp.maximum(jnp.dot(x, W1_ref[...],
                                preferred_element_type=jnp.float32)
                        + b1_ref[...][None, :], 0.0)
        x = jnp.maximum(jnp.dot(x, W2_ref[...],
                                preferred_element_type=jnp.float32)
                        + b2_ref[...][None, :], 0.0)
        x = jnp.maximum(jnp.dot(x, W3_ref[...],
                                preferred_element_type=jnp.float32)
                        + b3_ref[...][None, :], 0.0)
        y = jnp.concatenate([uu * vv, x], axis=1)
        out_ref[...] = (jnp.dot(y, Wo_ref[...],
                                preferred_element_type=jnp.float32)
                        + bo_ref[...][None, :])

    rep = lambda *shape: pl.BlockSpec(shape, lambda i: (0,) * len(shape))
    return pl.pallas_call(
        body,
        grid=grid,
        in_specs=[
            pl.BlockSpec((E, BS), lambda i: (0, i)),
            pl.BlockSpec((E, BS), lambda i: (0, i)),
            pl.BlockSpec((BS, E), lambda i: (i, 0)),
            pl.BlockSpec((H, BS), lambda i: (0, i)),
            rep(*W1.shape), rep(*b1.shape),
            rep(*W2.shape), rep(*b2.shape),
            rep(*W3.shape), rep(*b3.shape),
            rep(*W_out.shape), rep(*b_out.shape),
        ],
        out_specs=pl.BlockSpec((BS, 1), lambda i: (i, 0)),
        out_shape=jax.ShapeDtypeStruct((B, 1), jnp.float32),
    )(u_t, v_t, pooled, w_t, W1, b1, W2, b2, W3, b3, W_out, b_out)


def kernel(user_id, song_id, hist_ids, hist_weights, user_table, item_table,
           W1, b1, W2, b2, W3, b3, W_out, b_out):
    B, H = hist_ids.shape
    N, E = user_table.shape
    hist_flat = hist_ids.reshape(-1)
    hw_flat = hist_weights.reshape(-1)
    ut_t = user_table.T
    it_t = item_table.T
    w_t = hist_weights.T
    u_t, v_t = _sc_uvcols(ut_t, it_t, user_id, song_id)
    hist_flat, u_t = lax.optimization_barrier((hist_flat, u_t))
    pooled_flat = _sc_hist_item(item_table, hist_flat, hw_flat, B)
    pooled = pooled_flat.reshape(B, E)
    return _tc_mlp(u_t, v_t, pooled, w_t,
                   W1, b1, W2, b2, W3, b3, W_out, b_out)


# speedup vs baseline: 4.9127x; 1.0059x over previous
import functools

import jax
import jax.numpy as jnp
from jax import lax
from jax.experimental import pallas as pl
from jax.experimental.pallas import tpu as pltpu
from jax.experimental.pallas import tpu_sc as plsc

_NC = 2
_NS = 16
_NW = _NC * _NS
_L = 16


def _sc_hist_item(item_table, hist_flat, hw_flat, B):
    NH = hist_flat.shape[0]
    E = item_table.shape[1]
    H = NH // B
    b_per_w = B // _NW
    CB = 16
    CH = CB * H
    n_chunks = b_per_w // CB

    mesh = plsc.VectorSubcoreMesh(core_axis_name="c", subcore_axis_name="s")

    @functools.partial(
        pl.kernel,
        mesh=mesh,
        compiler_params=pltpu.CompilerParams(use_tc_tiling_on_sc=False,
                                             needs_layout_passes=False),
        out_type=jax.ShapeDtypeStruct((B * E,), jnp.float32),
        scratch_types=[
            pltpu.VMEM((CH,), jnp.int32),
            pltpu.VMEM((CH,), jnp.int32),
            pltpu.VMEM((CH, E), jnp.float32),
            pltpu.VMEM((CH, E), jnp.float32),
            pltpu.VMEM((b_per_w * H,), jnp.float32),
            pltpu.VMEM((b_per_w * E,), jnp.float32),
            pltpu.SemaphoreType.DMA,
            pltpu.SemaphoreType.DMA,
        ],
    )
    def hist_kernel(it_hbm, hid_hbm, hw_hbm, p_out,
                    idx_h0, idx_h1, rows_h0, rows_h1,
                    wv, pool_buf, sem0, sem1):
        wid = lax.axis_index("s") * _NC + lax.axis_index("c")
        base = wid * b_per_w
        hbase = base * H

        wcopy = pltpu.make_async_copy(
            hw_hbm.at[pl.ds(hbase, b_per_w * H)], wv, sem1)
        wcopy.start()

        col = [lax.iota(jnp.int32, _L) + k * _L for k in range(E // _L)]

        def start_gather(c, idx_h, rows_h, sem):
            off = hbase + c * CH
            pltpu.sync_copy(hid_hbm.at[pl.ds(off, CH)], idx_h)
            pltpu.make_async_copy(it_hbm.at[idx_h], rows_h, sem).start()

        def compute_chunk(c, idx_h, rows_h, sem):
            pltpu.make_async_copy(it_hbm.at[idx_h], rows_h, sem).wait()

            @pl.loop(0, CB)
            def _(b):
                rbase = b * H
                wbase = c * CH + rbase

                def jstep(j, acc):
                    rvec = jnp.full((_L,), rbase + j, dtype=jnp.int32)
                    wvec = plsc.load_gather(
                        wv, [jnp.full((_L,), wbase + j, dtype=jnp.int32)])
                    return tuple(
                        acc[k] + wvec * plsc.load_gather(rows_h, [rvec, col[k]])
                        for k in range(E // _L))

                acc = lax.fori_loop(
                    0, H, jstep,
                    tuple(jnp.zeros((_L,), jnp.float32)
                          for _ in range(E // _L)),
                    unroll=5)
                pbase = (c * CB + b) * E
                for k in range(E // _L):
                    pool_buf[pl.ds(pbase + k * _L, _L)] = acc[k]

        start_gather(0, idx_h0, rows_h0, sem0)
        wcopy.wait()

        @pl.loop(0, n_chunks // 2)
        def _(cc):
            c = cc * 2
            start_gather(c + 1, idx_h1, rows_h1, sem1)
            compute_chunk(c, idx_h0, rows_h0, sem0)

            @pl.when(c + 2 < n_chunks)
            def _():
                start_gather(c + 2, idx_h0, rows_h0, sem0)
            compute_chunk(c + 1, idx_h1, rows_h1, sem1)

        pltpu.sync_copy(pool_buf, p_out.at[pl.ds(base * E, b_per_w * E)])

    return hist_kernel(item_table, hist_flat, hw_flat)


def _sc_uvcols(ut_t, it_t, user_id, song_id):
    E, N = ut_t.shape
    B = user_id.shape[0]
    CPW = E // _NW

    mesh = plsc.VectorSubcoreMesh(core_axis_name="c", subcore_axis_name="s")

    @functools.partial(
        pl.kernel,
        mesh=mesh,
        compiler_params=pltpu.CompilerParams(needs_layout_passes=False),
        out_type=[
            jax.ShapeDtypeStruct((E, B), jnp.float32),
            jax.ShapeDtypeStruct((E, B), jnp.float32),
        ],
        scratch_types=[
            pltpu.VMEM((N,), jnp.float32),
            pltpu.VMEM((B,), jnp.int32),
            pltpu.VMEM((B,), jnp.int32),
            pltpu.VMEM((B,), jnp.float32),
        ],
    )
    def uvcol_kernel(ut_hbm, it_hbm, uid_hbm, sid_hbm, u_out, v_out,
                     col_buf, uid_v, sid_v, acc_v):
        wid = lax.axis_index("s") * _NC + lax.axis_index("c")
        pltpu.sync_copy(uid_hbm, uid_v)
        pltpu.sync_copy(sid_hbm, sid_v)

        def one_col(tab_hbm, ids_v, out_ref, c):
            pltpu.sync_copy(tab_hbm.at[c], col_buf)

            @pl.loop(0, B // _L)
            def _(g):
                idvec = ids_v[pl.ds(g * _L, _L)]
                acc_v[pl.ds(g * _L, _L)] = plsc.load_gather(col_buf, [idvec])
            pltpu.sync_copy(acc_v, out_ref.at[c])

        @pl.loop(0, CPW)
        def _(cc):
            c = wid * CPW + cc
            one_col(it_hbm, sid_v, v_out, c)
            one_col(ut_hbm, uid_v, u_out, c)

    return uvcol_kernel(ut_t, it_t, user_id, song_id)


def _tc_mlp(u_t, v_t, pooled, w_t, W1, b1, W2, b2, W3, b3, W_out, b_out):
    E, B = u_t.shape
    H = w_t.shape[0]
    BS = 1024
    grid = (B // BS,)

    def body(u_ref, v_ref, p_ref, w_ref,
             W1_ref, b1_ref, W2_ref, b2_ref, W3_ref, b3_ref,
             Wo_ref, bo_ref, out_ref):
        wsum = jnp.sum(w_ref[...], axis=0)[:, None] + 1e-8
        hist = p_ref[...] / wsum
        uu = u_ref[...].T
        vv = v_ref[...].T
        x = jnp.concatenate([uu, vv, hist], axis=1)
        x = jnp.maximum(jnp.dot(x, W1_ref[...],
                                preferred_element_type=jnp.float32)
                        + b1_ref[...][None, :], 0.0)
        x = jnp.maximum(jnp.dot(x, W2_ref[...],
                                preferred_element_type=jnp.float32)
                        + b2_ref[...][None, :], 0.0)
        x = jnp.maximum(jnp.dot(x, W3_ref[...],
                                preferred_element_type=jnp.float32)
                        + b3_ref[...][None, :], 0.0)
        y = jnp.concatenate([uu * vv, x], axis=1)
        out_ref[...] = (jnp.dot(y, Wo_ref[...],
                                preferred_element_type=jnp.float32)
                        + bo_ref[...][None, :])

    rep = lambda *shape: pl.BlockSpec(shape, lambda i: (0,) * len(shape))
    return pl.pallas_call(
        body,
        grid=grid,
        in_specs=[
            pl.BlockSpec((E, BS), lambda i: (0, i)),
            pl.BlockSpec((E, BS), lambda i: (0, i)),
            pl.BlockSpec((BS, E), lambda i: (i, 0)),
            pl.BlockSpec((H, BS), lambda i: (0, i)),
            rep(*W1.shape), rep(*b1.shape),
            rep(*W2.shape), rep(*b2.shape),
            rep(*W3.shape), rep(*b3.shape),
            rep(*W_out.shape), rep(*b_out.shape),
        ],
        out_specs=pl.BlockSpec((BS, 1), lambda i: (i, 0)),
        out_shape=jax.ShapeDtypeStruct((B, 1), jnp.float32),
    )(u_t, v_t, pooled, w_t, W1, b1, W2, b2, W3, b3, W_out, b_out)


def kernel(user_id, song_id, hist_ids, hist_weights, user_table, item_table,
           W1, b1, W2, b2, W3, b3, W_out, b_out):
    B, H = hist_ids.shape
    N, E = user_table.shape
    hist_flat = hist_ids.reshape(-1)
    hw_flat = hist_weights.reshape(-1)
    ut_t = user_table.T
    it_t = item_table.T
    w_t = hist_weights.T
    u_t, v_t = _sc_uvcols(ut_t, it_t, user_id, song_id)
    hist_flat, u_t = lax.optimization_barrier((hist_flat, u_t))
    pooled_flat = _sc_hist_item(item_table, hist_flat, hw_flat, B)
    pooled = pooled_flat.reshape(B, E)
    return _tc_mlp(u_t, v_t, pooled, w_t,
                   W1, b1, W2, b2, W3, b3, W_out, b_out)
